# baseline (speedup 1.0000x reference)
import functools
import math

import jax
import jax.numpy as jnp
from jax import lax
from jax.experimental import pallas as pl
from jax.experimental.pallas import tpu as pltpu
from jax.experimental.pallas import tpu_sc as plsc

_LANES = 16


def _build_gather(total, dim, num_heads, table_rows):
    info = plsc.get_sparse_core_info()
    nc, ns = info.num_cores, info.num_subcores
    nw = nc * ns
    per_w = total // nw
    assert per_w * nw == total

    period = (_LANES * num_heads) // math.gcd(_LANES, num_heads)
    n_pat = period // _LANES
    assert per_w % period == 0
    groups = per_w // period

    n_chunks = 8
    chunk = per_w // n_chunks
    assert chunk % 8 == 0 and chunk * n_chunks == per_w

    mesh = plsc.VectorSubcoreMesh(core_axis_name="c", subcore_axis_name="s")

    @functools.partial(
        pl.kernel,
        mesh=mesh,
        out_type=jax.ShapeDtypeStruct((total, dim), jnp.float32),
        compiler_params=pltpu.CompilerParams(use_tc_tiling_on_sc=False),
        scratch_types=[
            pltpu.VMEM((period,), jnp.int32),
            pltpu.VMEM((per_w,), jnp.int32),
            pltpu.VMEM((chunk, dim), jnp.float32),
            pltpu.VMEM((chunk, dim), jnp.float32),
            pltpu.SemaphoreType.DMA,
            pltpu.SemaphoreType.DMA,
            pltpu.SemaphoreType.DMA,
            pltpu.SemaphoreType.DMA,
        ],
    )
    def gather_kernel(hash_hbm, pat_hbm, table_hbm, out_hbm,
                      pat_v, ids_v, rows0, rows1, gsem0, gsem1, wsem0, wsem1):
        wid = lax.axis_index("s") * nc + lax.axis_index("c")
        base = wid * per_w

        pltpu.sync_copy(hash_hbm.at[pl.ds(base, per_w)], ids_v)
        pltpu.sync_copy(pat_hbm, pat_v)

        pats = [pat_v[pl.ds(j * _LANES, _LANES)] for j in range(n_pat)]

        def add_body(g, carry):
            s0 = g * period
            for j in range(n_pat):
                sl = pl.ds(s0 + j * _LANES, _LANES)
                ids_v[sl] = ids_v[sl] + pats[j]
            return carry

        lax.fori_loop(0, groups, add_body, 0)

        rows = (rows0, rows1)
        gsems = (gsem0, gsem1)
        wsems = (wsem0, wsem1)

        def gather_copy(k, buf):
            return pltpu.async_copy(
                table_hbm.at[ids_v.at[pl.ds(k * chunk, chunk)]],
                buf, gsems[k % 2])

        pending = [None, None]
        gather_copy(0, rows[0])
        for k in range(n_chunks):
            b = k % 2
            pltpu.make_async_copy(
                table_hbm.at[ids_v.at[pl.ds(k * chunk, chunk)]],
                rows[b], gsems[b]).wait()
            if k + 1 < n_chunks:
                nb = (k + 1) % 2
                if pending[nb] is not None:
                    pending[nb].wait()
                    pending[nb] = None
                gather_copy(k + 1, rows[nb])
            pending[b] = pltpu.async_copy(
                rows[b], out_hbm.at[pl.ds(base + k * chunk, chunk)], wsems[b])
        for p in pending:
            if p is not None:
                p.wait()

    return gather_kernel


def kernel(hash_ids, table, offsets):
    batch, num_heads = hash_ids.shape
    table_rows, dim = table.shape
    total = batch * num_heads
    flat_hash = hash_ids.reshape(total)
    period = (_LANES * num_heads) // math.gcd(_LANES, num_heads)
    pat = jnp.take(offsets, jnp.arange(period, dtype=jnp.int32) % num_heads)
    gk = _build_gather(total, dim, num_heads, table_rows)
    out = gk(flat_hash, pat, table)
    return out.reshape(batch, num_heads, dim)


# speedup vs baseline: 1.0301x; 1.0301x over previous
import functools

import jax
import jax.numpy as jnp
from jax import lax
from jax.experimental import pallas as pl
from jax.experimental.pallas import tpu as pltpu
from jax.experimental.pallas import tpu_sc as plsc

_LANES = 16


def _build_gather(batch, dim, num_heads):
    info = plsc.get_sparse_core_info()
    nc, ns = info.num_cores, info.num_subcores
    assert num_heads % nc == 0
    h_per_w = num_heads // nc
    b_per_w = batch // ns
    assert b_per_w * ns == batch and b_per_w % 8 == 0
    total = batch * num_heads
    vregs_per_row = b_per_w // _LANES
    assert vregs_per_row * _LANES == b_per_w

    mesh = plsc.VectorSubcoreMesh(core_axis_name="c", subcore_axis_name="s")

    @functools.partial(
        pl.kernel,
        mesh=mesh,
        out_type=jax.ShapeDtypeStruct((total, dim), jnp.float32),
        compiler_params=pltpu.CompilerParams(use_tc_tiling_on_sc=False),
        scratch_types=[
            pltpu.VMEM((num_heads * _LANES,), jnp.int32),
            pltpu.VMEM((h_per_w, b_per_w), jnp.int32),
            pltpu.VMEM((b_per_w, dim), jnp.float32),
            pltpu.VMEM((b_per_w, dim), jnp.float32),
            pltpu.SemaphoreType.DMA,
            pltpu.SemaphoreType.DMA,
            pltpu.SemaphoreType.DMA,
            pltpu.SemaphoreType.DMA,
        ],
    )
    def gather_kernel(hash_t_hbm, pat_hbm, table_hbm, out_hbm,
                      pat_v, ids_v, rows0, rows1, gsem0, gsem1, wsem0, wsem1):
        wh = lax.axis_index("c")
        wb = lax.axis_index("s")
        h0 = wh * h_per_w
        b0 = wb * b_per_w

        pltpu.sync_copy(hash_t_hbm.at[pl.ds(h0, h_per_w), pl.ds(b0, b_per_w)],
                        ids_v)
        pltpu.sync_copy(pat_hbm, pat_v)

        pats = [pat_v[pl.ds((h0 + hl) * _LANES, _LANES)]
                for hl in range(h_per_w)]

        def add_body(j, carry):
            s = j * _LANES
            for hl in range(h_per_w):
                ids_v[hl, pl.ds(s, _LANES)] = ids_v[hl, pl.ds(s, _LANES)] + pats[hl]
            return carry

        lax.fori_loop(0, vregs_per_row, add_body, 0)

        rows = (rows0, rows1)
        gsems = (gsem0, gsem1)
        wsems = (wsem0, wsem1)
        for hl in range(h_per_w):
            j = hl % 2
            if hl >= 2:
                pltpu.make_async_copy(
                    rows[j],
                    out_hbm.at[pl.ds((h0 + hl - 2) * batch + b0, b_per_w)],
                    wsems[j]).wait()
            pltpu.async_copy(
                table_hbm.at[ids_v.at[hl]], rows[j], gsems[j]).wait()
            pltpu.async_copy(
                rows[j],
                out_hbm.at[pl.ds((h0 + hl) * batch + b0, b_per_w)],
                wsems[j])
        for hl in (h_per_w - 2, h_per_w - 1):
            j = hl % 2
            pltpu.make_async_copy(
                rows[j],
                out_hbm.at[pl.ds((h0 + hl) * batch + b0, b_per_w)],
                wsems[j]).wait()

    return gather_kernel


def kernel(hash_ids, table, offsets):
    batch, num_heads = hash_ids.shape
    table_rows, dim = table.shape
    hash_t = hash_ids.T
    pat = jnp.repeat(offsets, _LANES)
    gk = _build_gather(batch, dim, num_heads)
    out_t = gk(hash_t, pat, table)
    return out_t.reshape(num_heads, batch, dim).transpose(1, 0, 2)


# speedup vs baseline: 1.6290x; 1.5815x over previous
import functools

import jax
import jax.numpy as jnp
from jax import lax
from jax.experimental import pallas as pl
from jax.experimental.pallas import tpu as pltpu
from jax.experimental.pallas import tpu_sc as plsc

_LANES = 16


def _build_gather(batch, dim, num_heads, s_rows, pack):
    info = plsc.get_sparse_core_info()
    nc, ns = info.num_cores, info.num_subcores
    assert num_heads % nc == 0
    h_per_w = num_heads // nc
    b_per_w = batch // ns
    assert b_per_w * ns == batch and b_per_w % 8 == 0
    total = batch * num_heads
    vregs_per_row = b_per_w // _LANES
    assert vregs_per_row * _LANES == b_per_w

    mesh = plsc.VectorSubcoreMesh(core_axis_name="c", subcore_axis_name="s")

    @functools.partial(
        pl.kernel,
        mesh=mesh,
        out_type=jax.ShapeDtypeStruct((total, dim), jnp.float32),
        compiler_params=pltpu.CompilerParams(use_tc_tiling_on_sc=False),
        scratch_types=[
            pltpu.VMEM((num_heads * _LANES,), jnp.int32),
            pltpu.VMEM((h_per_w, b_per_w), jnp.int32),
            pltpu.VMEM((b_per_w, dim), jnp.float32),
            pltpu.VMEM((b_per_w, dim), jnp.float32),
            pltpu.SemaphoreType.DMA,
            pltpu.SemaphoreType.DMA,
            pltpu.SemaphoreType.DMA,
            pltpu.SemaphoreType.DMA,
        ],
    )
    def gather_kernel(hash_t_hbm, pat_hbm, table_hbm, out_hbm,
                      pat_v, ids_v, rows0, rows1, gsem0, gsem1, wsem0, wsem1):
        wh = lax.axis_index("c")
        wb = lax.axis_index("s")
        h0 = wh * h_per_w
        b0 = wb * b_per_w

        pltpu.sync_copy(hash_t_hbm.at[pl.ds(h0, h_per_w), pl.ds(b0, b_per_w)],
                        ids_v)
        pltpu.sync_copy(pat_hbm, pat_v)

        pats = [pat_v[pl.ds((h0 + hl) * _LANES, _LANES)]
                for hl in range(h_per_w)]

        def add_body(j, carry):
            s = j * _LANES
            for hl in range(h_per_w):
                r = ids_v[hl, pl.ds(s, _LANES)] + pats[hl]
                k = lax.div(r, jnp.int32(s_rows))
                ids_v[hl, pl.ds(s, _LANES)] = (
                    r * pack - k * jnp.int32(pack * s_rows - 1))
            return carry

        lax.fori_loop(0, vregs_per_row, add_body, 0)

        rows = (rows0, rows1)
        gsems = (gsem0, gsem1)
        wsems = (wsem0, wsem1)
        for hl in range(h_per_w):
            j = hl % 2
            if hl >= 2:
                pltpu.make_async_copy(
                    rows[j],
                    out_hbm.at[pl.ds((h0 + hl - 2) * batch + b0, b_per_w)],
                    wsems[j]).wait()
            pltpu.async_copy(
                table_hbm.at[ids_v.at[hl]], rows[j], gsems[j]).wait()
            pltpu.async_copy(
                rows[j],
                out_hbm.at[pl.ds((h0 + hl) * batch + b0, b_per_w)],
                wsems[j])
        for hl in (h_per_w - 2, h_per_w - 1):
            j = hl % 2
            pltpu.make_async_copy(
                rows[j],
                out_hbm.at[pl.ds((h0 + hl) * batch + b0, b_per_w)],
                wsems[j]).wait()

    return gather_kernel


def _tc_detranspose(table_t, dim, col_block, n_grid):
    pack = 128 // dim
    s_rows = n_grid * col_block

    def body(*refs):
        xs, o_ref = refs[:-1], refs[-1]
        for k in range(pack):
            o_ref[:, k * dim:(k + 1) * dim] = jnp.swapaxes(xs[k][...], 0, 1)

    max_blk = (table_t.shape[1] - 1) // col_block
    specs = [
        pl.BlockSpec(
            (dim, col_block),
            functools.partial(
                lambda k, j: (0, jnp.minimum(j + k * n_grid, max_blk)), k))
        for k in range(pack)
    ]
    return pl.pallas_call(
        body,
        grid=(n_grid,),
        in_specs=specs,
        out_specs=pl.BlockSpec((col_block, 128), lambda j: (j, 0)),
        out_shape=jax.ShapeDtypeStruct((s_rows, 128), jnp.float32),
    )(*([table_t] * pack))


def kernel(hash_ids, table, offsets):
    batch, num_heads = hash_ids.shape
    table_rows, dim = table.shape
    hash_t = hash_ids.T
    pat = jnp.repeat(offsets, _LANES)
    pack = 128 // dim
    col_block = 2048
    n_grid = -(-table_rows // (pack * col_block))
    s_rows = n_grid * col_block
    table_c = _tc_detranspose(table.T, dim, col_block, n_grid)
    table_c = table_c.reshape(pack * s_rows, dim)
    gk = _build_gather(batch, dim, num_heads, s_rows, pack)
    out_t = gk(hash_t, pat, table_c)
    return out_t.reshape(num_heads, batch, dim).transpose(1, 0, 2)


# speedup vs baseline: 3.0515x; 1.8733x over previous
import functools

import jax
import jax.numpy as jnp
from jax import lax
from jax.experimental import pallas as pl
from jax.experimental.pallas import tpu as pltpu
from jax.experimental.pallas import tpu_sc as plsc

_LANES = 16


def _build_gather(batch, dim, num_heads, s_rows, pack):
    info = plsc.get_sparse_core_info()
    nc, ns = info.num_cores, info.num_subcores
    assert num_heads % nc == 0
    h_per_w = num_heads // nc
    b_per_w = batch // ns
    assert b_per_w * ns == batch and b_per_w % 8 == 0
    total = batch * num_heads
    vregs_per_row = b_per_w // _LANES
    assert vregs_per_row * _LANES == b_per_w

    mesh = plsc.VectorSubcoreMesh(core_axis_name="c", subcore_axis_name="s")

    @functools.partial(
        pl.kernel,
        mesh=mesh,
        out_type=jax.ShapeDtypeStruct((total, dim), jnp.float32),
        compiler_params=pltpu.CompilerParams(use_tc_tiling_on_sc=False),
        scratch_types=[
            pltpu.VMEM((num_heads * _LANES,), jnp.int32),
            pltpu.VMEM((num_heads * _LANES,), jnp.int32),
            pltpu.VMEM((h_per_w, b_per_w), jnp.int32),
            pltpu.VMEM((b_per_w, dim), jnp.float32),
            pltpu.VMEM((b_per_w, dim), jnp.float32),
            pltpu.SemaphoreType.DMA,
            pltpu.SemaphoreType.DMA,
            pltpu.SemaphoreType.DMA,
            pltpu.SemaphoreType.DMA,
        ],
    )
    def gather_kernel(hash_t_hbm, a_hbm, c_hbm, table_hbm, out_hbm,
                      a_v, c_v, ids_v, rows0, rows1,
                      gsem0, gsem1, wsem0, wsem1):
        wh = lax.axis_index("c")
        wb = lax.axis_index("s")
        h0 = wh * h_per_w
        b0 = wb * b_per_w

        pltpu.sync_copy(hash_t_hbm.at[pl.ds(h0, h_per_w), pl.ds(b0, b_per_w)],
                        ids_v)
        pltpu.sync_copy(a_hbm, a_v)
        pltpu.sync_copy(c_hbm, c_v)

        pat_a = [a_v[pl.ds((h0 + hl) * _LANES, _LANES)]
                 for hl in range(h_per_w)]
        pat_c = [c_v[pl.ds((h0 + hl) * _LANES, _LANES)]
                 for hl in range(h_per_w)]
        wrap = jnp.full((_LANES,), pack * s_rows - 1, jnp.int32)

        def add_body(j, carry):
            s = j * _LANES
            for hl in range(h_per_w):
                hsh = ids_v[hl, pl.ds(s, _LANES)]
                idx = hsh * pack + pat_a[hl]
                ids_v[hl, pl.ds(s, _LANES)] = jnp.where(
                    hsh >= pat_c[hl], idx - wrap, idx)
            return carry

        lax.fori_loop(0, vregs_per_row, add_body, 0)

        rows = (rows0, rows1)
        gsems = (gsem0, gsem1)
        wsems = (wsem0, wsem1)
        for hl in range(h_per_w):
            j = hl % 2
            if hl >= 2:
                pltpu.make_async_copy(
                    rows[j],
                    out_hbm.at[pl.ds((h0 + hl - 2) * batch + b0, b_per_w)],
                    wsems[j]).wait()
            pltpu.async_copy(
                table_hbm.at[ids_v.at[hl]], rows[j], gsems[j]).wait()
            pltpu.async_copy(
                rows[j],
                out_hbm.at[pl.ds((h0 + hl) * batch + b0, b_per_w)],
                wsems[j])
        for hl in (h_per_w - 2, h_per_w - 1):
            j = hl % 2
            pltpu.make_async_copy(
                rows[j],
                out_hbm.at[pl.ds((h0 + hl) * batch + b0, b_per_w)],
                wsems[j]).wait()

    return gather_kernel


def _tc_detranspose(table_t, dim, col_block, n_grid):
    pack = 128 // dim
    s_rows = n_grid * col_block

    def body(*refs):
        xs, o_ref = refs[:-1], refs[-1]
        stacked = jnp.concatenate([x[...] for x in xs], axis=0)
        o_ref[...] = jnp.swapaxes(stacked, 0, 1)

    max_blk = (table_t.shape[1] - 1) // col_block
    specs = [
        pl.BlockSpec(
            (dim, col_block),
            functools.partial(
                lambda k, j: (0, jnp.minimum(j + k * n_grid, max_blk)), k))
        for k in range(pack)
    ]
    return pl.pallas_call(
        body,
        grid=(n_grid,),
        in_specs=specs,
        out_specs=pl.BlockSpec((col_block, 128), lambda j: (j, 0)),
        out_shape=jax.ShapeDtypeStruct((s_rows, 128), jnp.float32),
    )(*([table_t] * pack))


def kernel(hash_ids, table, offsets):
    batch, num_heads = hash_ids.shape
    table_rows, dim = table.shape
    hash_t = hash_ids.T
    pack = 128 // dim
    col_block = 4096
    n_grid = -(-table_rows // (pack * col_block))
    s_rows = n_grid * col_block
    table_c = _tc_detranspose(table.T, dim, col_block, n_grid)
    table_c = table_c.reshape(pack * s_rows, dim)
    k0 = offsets // s_rows
    a_pat = jnp.repeat(pack * offsets - (pack * s_rows - 1) * k0, _LANES)
    c_pat = jnp.repeat(s_rows * (k0 + 1) - offsets, _LANES)
    gk = _build_gather(batch, dim, num_heads, s_rows, pack)
    out_t = gk(hash_t, a_pat, c_pat, table_c)
    return out_t.reshape(num_heads, batch, dim).transpose(1, 0, 2)


# speedup vs baseline: 3.2235x; 1.0564x over previous
import functools

import jax
import jax.numpy as jnp
from jax import lax
from jax.experimental import pallas as pl
from jax.experimental.pallas import tpu as pltpu
from jax.experimental.pallas import tpu_sc as plsc

_LANES = 16


def _build_gather(batch, dim, num_heads, s_rows, pack):
    info = plsc.get_sparse_core_info()
    nc, ns = info.num_cores, info.num_subcores
    assert num_heads % nc == 0
    h_per_w = num_heads // nc
    b_per_w = batch // ns
    assert b_per_w * ns == batch and b_per_w % 8 == 0
    total = batch * num_heads
    vregs_per_row = b_per_w // _LANES
    assert vregs_per_row * _LANES == b_per_w

    mesh = plsc.VectorSubcoreMesh(core_axis_name="c", subcore_axis_name="s")

    @functools.partial(
        pl.kernel,
        mesh=mesh,
        out_type=jax.ShapeDtypeStruct((total, dim), jnp.float32),
        compiler_params=pltpu.CompilerParams(use_tc_tiling_on_sc=False),
        scratch_types=[
            pltpu.VMEM((num_heads * _LANES,), jnp.int32),
            pltpu.VMEM((num_heads * _LANES,), jnp.int32),
            pltpu.VMEM((h_per_w, b_per_w), jnp.int32),
            pltpu.VMEM((b_per_w, dim), jnp.float32),
            pltpu.VMEM((b_per_w, dim), jnp.float32),
            pltpu.SemaphoreType.DMA,
            pltpu.SemaphoreType.DMA,
            pltpu.SemaphoreType.DMA,
            pltpu.SemaphoreType.DMA,
        ],
    )
    def gather_kernel(hash_t_hbm, a_hbm, c_hbm, table_hbm, out_hbm,
                      a_v, c_v, ids_v, rows0, rows1,
                      gsem0, gsem1, wsem0, wsem1):
        wh = lax.axis_index("c")
        wb = lax.axis_index("s")
        h0 = wh * h_per_w
        b0 = wb * b_per_w

        pltpu.sync_copy(hash_t_hbm.at[pl.ds(h0, h_per_w), pl.ds(b0, b_per_w)],
                        ids_v)
        pltpu.sync_copy(a_hbm, a_v)
        pltpu.sync_copy(c_hbm, c_v)

        pat_a = [a_v[pl.ds((h0 + hl) * _LANES, _LANES)]
                 for hl in range(h_per_w)]
        pat_c = [c_v[pl.ds((h0 + hl) * _LANES, _LANES)]
                 for hl in range(h_per_w)]
        wrap = jnp.full((_LANES,), pack * s_rows - 1, jnp.int32)

        def add_body(j, carry):
            s = j * _LANES
            for hl in range(h_per_w):
                hsh = ids_v[hl, pl.ds(s, _LANES)]
                idx = hsh * pack + pat_a[hl]
                ids_v[hl, pl.ds(s, _LANES)] = jnp.where(
                    hsh >= pat_c[hl], idx - wrap, idx)
            return carry

        lax.fori_loop(0, vregs_per_row, add_body, 0)

        rows = (rows0, rows1)
        gsems = (gsem0, gsem1)
        wsems = (wsem0, wsem1)
        for hl in range(h_per_w):
            j = hl % 2
            if hl >= 2:
                pltpu.make_async_copy(
                    rows[j],
                    out_hbm.at[pl.ds((h0 + hl - 2) * batch + b0, b_per_w)],
                    wsems[j]).wait()
            pltpu.async_copy(
                table_hbm.at[ids_v.at[hl]], rows[j], gsems[j]).wait()
            pltpu.async_copy(
                rows[j],
                out_hbm.at[pl.ds((h0 + hl) * batch + b0, b_per_w)],
                wsems[j])
        for hl in (h_per_w - 2, h_per_w - 1):
            j = hl % 2
            pltpu.make_async_copy(
                rows[j],
                out_hbm.at[pl.ds((h0 + hl) * batch + b0, b_per_w)],
                wsems[j]).wait()

    return gather_kernel


def _tc_detranspose(table_t, dim, col_block, n_grid):
    pack = 128 // dim
    s_rows = n_grid * col_block

    def body(*refs):
        xs, o_ref = refs[:-1], refs[-1]
        stacked = jnp.concatenate([x[...] for x in xs], axis=0)
        o_ref[...] = jnp.swapaxes(stacked, 0, 1)

    max_blk = (table_t.shape[1] - 1) // col_block
    specs = [
        pl.BlockSpec(
            (dim, col_block),
            functools.partial(
                lambda k, j: (0, jnp.minimum(j + k * n_grid, max_blk)), k))
        for k in range(pack)
    ]
    return pl.pallas_call(
        body,
        grid=(n_grid,),
        in_specs=specs,
        out_specs=pl.BlockSpec((col_block, 128), lambda j: (j, 0)),
        out_shape=jax.ShapeDtypeStruct((s_rows, 128), jnp.float32),
    )(*([table_t] * pack))


def kernel(hash_ids, table, offsets):
    batch, num_heads = hash_ids.shape
    table_rows, dim = table.shape
    hash_t = hash_ids.T
    pack = 128 // dim
    col_block = 8192
    n_grid = -(-table_rows // (pack * col_block))
    s_rows = n_grid * col_block
    table_c = _tc_detranspose(table.T, dim, col_block, n_grid)
    table_c = table_c.reshape(pack * s_rows, dim)
    k0 = offsets // s_rows
    a_pat = jnp.repeat(pack * offsets - (pack * s_rows - 1) * k0, _LANES)
    c_pat = jnp.repeat(s_rows * (k0 + 1) - offsets, _LANES)
    gk = _build_gather(batch, dim, num_heads, s_rows, pack)
    out_t = gk(hash_t, a_pat, c_pat, table_c)
    return out_t.reshape(num_heads, batch, dim).transpose(1, 0, 2)


# speedup vs baseline: 4.6097x; 1.4300x over previous
import functools

import jax
import jax.numpy as jnp
from jax import lax
from jax.experimental import pallas as pl
from jax.experimental.pallas import tpu as pltpu
from jax.experimental.pallas import tpu_sc as plsc

_LANES = 16


def _build_gather(batch, dim, num_heads, s_rows, pack):
    info = plsc.get_sparse_core_info()
    nc, ns = info.num_cores, info.num_subcores
    assert num_heads % nc == 0
    h_per_w = num_heads // nc
    b_per_w = batch // ns
    assert b_per_w * ns == batch and b_per_w % 8 == 0
    total = batch * num_heads
    vregs_per_row = b_per_w // _LANES
    assert vregs_per_row * _LANES == b_per_w

    mesh = plsc.VectorSubcoreMesh(core_axis_name="c", subcore_axis_name="s")

    @functools.partial(
        pl.kernel,
        mesh=mesh,
        out_type=jax.ShapeDtypeStruct((total, 128), jnp.float32),
        compiler_params=pltpu.CompilerParams(use_tc_tiling_on_sc=False),
        scratch_types=[
            pltpu.VMEM((num_heads * _LANES,), jnp.int32),
            pltpu.VMEM((num_heads * _LANES,), jnp.int32),
            pltpu.VMEM((h_per_w, b_per_w), jnp.int32),
            pltpu.VMEM((b_per_w, dim), jnp.float32),
            pltpu.VMEM((b_per_w, dim), jnp.float32),
            pltpu.SemaphoreType.DMA,
            pltpu.SemaphoreType.DMA,
            pltpu.SemaphoreType.DMA,
            pltpu.SemaphoreType.DMA,
        ],
    )
    def gather_kernel(hash_t_hbm, a_hbm, c_hbm, table_hbm, out_hbm,
                      a_v, c_v, ids_v, rows0, rows1,
                      gsem0, gsem1, wsem0, wsem1):
        wh = lax.axis_index("c")
        wb = lax.axis_index("s")
        h0 = wh * h_per_w
        b0 = wb * b_per_w

        pltpu.sync_copy(hash_t_hbm.at[pl.ds(h0, h_per_w), pl.ds(b0, b_per_w)],
                        ids_v)
        pltpu.sync_copy(a_hbm, a_v)
        pltpu.sync_copy(c_hbm, c_v)

        pat_a = [a_v[pl.ds((h0 + hl) * _LANES, _LANES)]
                 for hl in range(h_per_w)]
        pat_c = [c_v[pl.ds((h0 + hl) * _LANES, _LANES)]
                 for hl in range(h_per_w)]
        wrap = jnp.full((_LANES,), pack * s_rows - 1, jnp.int32)

        def add_body(j, carry):
            s = j * _LANES
            for hl in range(h_per_w):
                hsh = ids_v[hl, pl.ds(s, _LANES)]
                idx = hsh * pack + pat_a[hl]
                ids_v[hl, pl.ds(s, _LANES)] = jnp.where(
                    hsh >= pat_c[hl], idx - wrap, idx)
            return carry

        lax.fori_loop(0, vregs_per_row, add_body, 0)

        rows = (rows0, rows1)
        gsems = (gsem0, gsem1)
        wsems = (wsem0, wsem1)
        for hl in range(h_per_w):
            j = hl % 2
            if hl >= 2:
                pltpu.make_async_copy(
                    rows[j],
                    out_hbm.at[pl.ds((h0 + hl - 2) * batch + b0, b_per_w),
                               pl.ds(0, dim)],
                    wsems[j]).wait()
            pltpu.async_copy(
                table_hbm.at[ids_v.at[hl]], rows[j], gsems[j]).wait()
            pltpu.async_copy(
                rows[j],
                out_hbm.at[pl.ds((h0 + hl) * batch + b0, b_per_w),
                           pl.ds(0, dim)],
                wsems[j])
        for hl in (h_per_w - 2, h_per_w - 1):
            j = hl % 2
            pltpu.make_async_copy(
                rows[j],
                out_hbm.at[pl.ds((h0 + hl) * batch + b0, b_per_w),
                           pl.ds(0, dim)],
                wsems[j]).wait()

    return gather_kernel


def _tc_detranspose(table_t, dim, col_block, n_grid):
    pack = 128 // dim
    s_rows = n_grid * col_block

    def body(*refs):
        xs, o_ref = refs[:-1], refs[-1]
        stacked = jnp.concatenate([x[...] for x in xs], axis=0)
        o_ref[...] = jnp.swapaxes(stacked, 0, 1)

    max_blk = (table_t.shape[1] - 1) // col_block
    specs = [
        pl.BlockSpec(
            (dim, col_block),
            functools.partial(
                lambda k, j: (0, jnp.minimum(j + k * n_grid, max_blk)), k))
        for k in range(pack)
    ]
    return pl.pallas_call(
        body,
        grid=(n_grid,),
        in_specs=specs,
        out_specs=pl.BlockSpec((col_block, 128), lambda j: (j, 0)),
        out_shape=jax.ShapeDtypeStruct((s_rows, 128), jnp.float32),
    )(*([table_t] * pack))


def kernel(hash_ids, table, offsets):
    batch, num_heads = hash_ids.shape
    table_rows, dim = table.shape
    hash_t = hash_ids.T
    pack = 128 // dim
    col_block = 8192
    n_grid = -(-table_rows // (pack * col_block))
    s_rows = n_grid * col_block
    table_c = _tc_detranspose(table.T, dim, col_block, n_grid)
    table_c = table_c.reshape(pack * s_rows, dim)
    k0 = offsets // s_rows
    a_pat = jnp.repeat(pack * offsets - (pack * s_rows - 1) * k0, _LANES)
    c_pat = jnp.repeat(s_rows * (k0 + 1) - offsets, _LANES)
    gk = _build_gather(batch, dim, num_heads, s_rows, pack)
    out_t = gk(hash_t, a_pat, c_pat, table_c)
    out_t = out_t[:, :dim].reshape(num_heads, batch, dim)
    return out_t.transpose(1, 0, 2)


# speedup vs baseline: 4.6969x; 1.0189x over previous
import functools

import jax
import jax.numpy as jnp
from jax import lax
from jax.experimental import pallas as pl
from jax.experimental.pallas import tpu as pltpu
from jax.experimental.pallas import tpu_sc as plsc

_LANES = 16


def _build_gather(batch, dim, num_heads, s_rows, pack):
    info = plsc.get_sparse_core_info()
    nc, ns = info.num_cores, info.num_subcores
    assert num_heads % nc == 0
    h_per_w = num_heads // nc
    b_per_w = batch // ns
    assert b_per_w * ns == batch and b_per_w % 8 == 0
    total = batch * num_heads
    vregs_per_row = b_per_w // _LANES
    assert vregs_per_row * _LANES == b_per_w

    mesh = plsc.VectorSubcoreMesh(core_axis_name="c", subcore_axis_name="s")

    @functools.partial(
        pl.kernel,
        mesh=mesh,
        out_type=jax.ShapeDtypeStruct((total, 128), jnp.float32),
        compiler_params=pltpu.CompilerParams(use_tc_tiling_on_sc=False),
        scratch_types=[
            pltpu.VMEM((num_heads * _LANES,), jnp.int32),
            pltpu.VMEM((num_heads * _LANES,), jnp.int32),
            pltpu.VMEM((h_per_w, b_per_w), jnp.int32),
            pltpu.VMEM((b_per_w, dim), jnp.float32),
            pltpu.VMEM((b_per_w, dim), jnp.float32),
            pltpu.SemaphoreType.DMA,
            pltpu.SemaphoreType.DMA,
            pltpu.SemaphoreType.DMA,
            pltpu.SemaphoreType.DMA,
        ],
    )
    def gather_kernel(hash_t_hbm, a_hbm, c_hbm, table_hbm, out_hbm,
                      a_v, c_v, ids_v, rows0, rows1,
                      gsem0, gsem1, wsem0, wsem1):
        wh = lax.axis_index("c")
        wb = lax.axis_index("s")
        h0 = wh * h_per_w
        b0 = wb * b_per_w

        pltpu.sync_copy(hash_t_hbm.at[pl.ds(h0, h_per_w), pl.ds(b0, b_per_w)],
                        ids_v)
        pltpu.sync_copy(a_hbm, a_v)
        pltpu.sync_copy(c_hbm, c_v)

        pat_a = [a_v[pl.ds((h0 + hl) * _LANES, _LANES)]
                 for hl in range(h_per_w)]
        pat_c = [c_v[pl.ds((h0 + hl) * _LANES, _LANES)]
                 for hl in range(h_per_w)]
        wrap = jnp.full((_LANES,), pack * s_rows - 1, jnp.int32)

        def add_body(j, carry):
            s = j * _LANES
            for hl in range(h_per_w):
                hsh = ids_v[hl, pl.ds(s, _LANES)]
                idx = hsh * pack + pat_a[hl]
                ids_v[hl, pl.ds(s, _LANES)] = jnp.where(
                    hsh >= pat_c[hl], idx - wrap, idx)
            return carry

        lax.fori_loop(0, vregs_per_row, add_body, 0)

        rows = (rows0, rows1)
        gsems = (gsem0, gsem1)
        wsems = (wsem0, wsem1)
        for hl in range(h_per_w):
            j = hl % 2
            if hl >= 2:
                pltpu.make_async_copy(
                    rows[j],
                    out_hbm.at[pl.ds((h0 + hl - 2) * batch + b0, b_per_w),
                               pl.ds(0, dim)],
                    wsems[j]).wait()
            pltpu.async_copy(
                table_hbm.at[ids_v.at[hl]], rows[j], gsems[j]).wait()
            pltpu.async_copy(
                rows[j],
                out_hbm.at[pl.ds((h0 + hl) * batch + b0, b_per_w),
                           pl.ds(0, dim)],
                wsems[j])
        for hl in (h_per_w - 2, h_per_w - 1):
            j = hl % 2
            pltpu.make_async_copy(
                rows[j],
                out_hbm.at[pl.ds((h0 + hl) * batch + b0, b_per_w),
                           pl.ds(0, dim)],
                wsems[j]).wait()

    return gather_kernel


def _tc_detranspose(table_t, dim, col_block, n_grid):
    pack = 128 // dim
    s_rows = n_grid * col_block

    def body(*refs):
        xs, o_ref = refs[:-1], refs[-1]
        stacked = jnp.concatenate([x[...] for x in xs], axis=0)
        o_ref[...] = jnp.swapaxes(stacked, 0, 1)

    max_blk = (table_t.shape[1] - 1) // col_block
    specs = [
        pl.BlockSpec(
            (dim, col_block),
            functools.partial(
                lambda k, j: (0, jnp.minimum(j + k * n_grid, max_blk)), k))
        for k in range(pack)
    ]
    return pl.pallas_call(
        body,
        grid=(n_grid,),
        in_specs=specs,
        out_specs=pl.BlockSpec((col_block, 128), lambda j: (j, 0)),
        out_shape=jax.ShapeDtypeStruct((s_rows, 128), jnp.float32),
    )(*([table_t] * pack))


def kernel(hash_ids, table, offsets):
    batch, num_heads = hash_ids.shape
    table_rows, dim = table.shape
    hash_t = hash_ids.T
    pack = 128 // dim
    col_block = 16384
    n_grid = -(-table_rows // (pack * col_block))
    s_rows = n_grid * col_block
    table_c = _tc_detranspose(table.T, dim, col_block, n_grid)
    table_c = table_c.reshape(pack * s_rows, dim)
    k0 = offsets // s_rows
    a_pat = jnp.repeat(pack * offsets - (pack * s_rows - 1) * k0, _LANES)
    c_pat = jnp.repeat(s_rows * (k0 + 1) - offsets, _LANES)
    gk = _build_gather(batch, dim, num_heads, s_rows, pack)
    out_t = gk(hash_t, a_pat, c_pat, table_c)
    out_t = out_t[:, :dim].reshape(num_heads, batch, dim)
    return out_t.transpose(1, 0, 2)
